# Initial kernel scaffold; baseline (speedup 1.0000x reference)
import jax
import jax.numpy as jnp
from jax.experimental import pallas as pl


def kernel(h, x, edge_index, edge_attr, W_e1, b_e1, W_e2, b_e2, W_n1, b_n1, W_n2, b_n2, W_c1, b_c1, W_c2, W_a, b_a):
    raise NotImplementedError("write your pallas kernel here")


# speedup vs baseline: 3.0535x; 3.0535x over previous
import functools

import jax
import jax.numpy as jnp
from jax import lax
from jax.experimental import pallas as pl
from jax.experimental.pallas import tpu as pltpu
from jax.experimental.pallas import tpu_sc as plsc

N = 10000
E = 320000
D = 128
XP = 16
TW = D + XP

NC, NS, L = 2, 16, 16
NW = NC * NS
EPW = E // NW
CH = 80
NCHUNK = EPW // CH
RPS = N // NS

_f32 = jnp.float32
_mesh = plsc.VectorSubcoreMesh(core_axis_name="c", subcore_axis_name="s")
_sc_params = pltpu.CompilerParams(use_tc_tiling_on_sc=False)


def _tables_body(h_ref, xp_ref, whr_ref, whc_ref, tr_ref, tc_ref):
    h = h_ref[...]
    xp = xp_ref[...]
    tr_ref[...] = jnp.concatenate(
        [jnp.dot(h, whr_ref[...], preferred_element_type=_f32), xp], axis=1)
    tc_ref[...] = jnp.concatenate(
        [jnp.dot(h, whc_ref[...], preferred_element_type=_f32), xp], axis=1)


def _make_tables(h, xp, whr, whc):
    nb = 1000
    grid = N // nb
    return pl.pallas_call(
        _tables_body,
        grid=(grid,),
        in_specs=[
            pl.BlockSpec((nb, D), lambda i: (i, 0)),
            pl.BlockSpec((nb, XP), lambda i: (i, 0)),
            pl.BlockSpec((D, D), lambda i: (0, 0)),
            pl.BlockSpec((D, D), lambda i: (0, 0)),
        ],
        out_specs=[
            pl.BlockSpec((nb, TW), lambda i: (i, 0)),
            pl.BlockSpec((nb, TW), lambda i: (i, 0)),
        ],
        out_shape=[
            jax.ShapeDtypeStruct((N, TW), _f32),
            jax.ShapeDtypeStruct((N, TW), _f32),
        ],
    )(h, xp, whr, whc)


@functools.partial(
    pl.kernel,
    out_type=(jax.ShapeDtypeStruct((E, D), _f32),
              jax.ShapeDtypeStruct((E, XP), _f32)),
    mesh=_mesh,
    scratch_types=[
        pltpu.VMEM((CH,), jnp.int32),
        pltpu.VMEM((CH,), jnp.int32),
        pltpu.VMEM((CH, TW), _f32),
        pltpu.VMEM((CH, TW), _f32),
        pltpu.VMEM((CH, D), _f32),
        pltpu.VMEM((CH, XP), _f32),
        pltpu.SemaphoreType.DMA,
        pltpu.SemaphoreType.DMA,
    ],
    compiler_params=_sc_params,
)
def _sc_gather(tr_hbm, tc_hbm, row_hbm, col_hbm, g_hbm, d_hbm,
               idxr, idxc, abuf, bbuf, gbuf, dbuf, sem_a, sem_b):
    wid = lax.axis_index("s") * NC + lax.axis_index("c")

    @pl.loop(0, NCHUNK)
    def _chunk(ci):
        base = wid * EPW + ci * CH
        pltpu.sync_copy(row_hbm.at[pl.ds(base, CH)], idxr)
        pltpu.sync_copy(col_hbm.at[pl.ds(base, CH)], idxc)
        cp_a = pltpu.async_copy(tr_hbm.at[idxr], abuf, sem_a)
        cp_b = pltpu.async_copy(tc_hbm.at[idxc], bbuf, sem_b)
        cp_a.wait()
        cp_b.wait()

        @pl.loop(0, CH)
        def _row(i):
            for j in range(D // L):
                sl = pl.ds(j * L, L)
                gbuf[i, sl] = abuf[i, sl] + bbuf[i, sl]
            xs = pl.ds(D, L)
            dbuf[i, pl.ds(0, L)] = abuf[i, xs] - bbuf[i, xs]

        pltpu.sync_copy(gbuf, g_hbm.at[pl.ds(base, CH)])
        pltpu.sync_copy(dbuf, d_hbm.at[pl.ds(base, CH)])


def _edge_body(g_ref, d_ref, ea_ref, wea_ref, wrad_ref, be1_ref, we2_ref,
               be2_ref, wa_ref, ba_ref, wc1_ref, bc1_ref, wc2_ref,
               m_ref, cv_ref):
    g = g_ref[...]
    d = d_ref[...]
    ea = ea_ref[...]
    radial = jnp.sum(d * d, axis=1, keepdims=True)
    pre = (g + jnp.dot(ea, wea_ref[...], preferred_element_type=_f32)
           + radial * wrad_ref[...] + be1_ref[...])
    m1 = jax.nn.silu(pre)
    m2 = jax.nn.silu(jnp.dot(m1, we2_ref[...], preferred_element_type=_f32)
                     + be2_ref[...])
    att = jax.nn.sigmoid(jnp.dot(m2, wa_ref[...], preferred_element_type=_f32)
                         + ba_ref[...])
    m = m2 * att
    m_ref[...] = m
    cu = jnp.dot(jax.nn.silu(jnp.dot(m, wc1_ref[...],
                                     preferred_element_type=_f32)
                             + bc1_ref[...]),
                 wc2_ref[...], preferred_element_type=_f32)
    cv = d * cu
    lane = lax.broadcasted_iota(jnp.int32, cv.shape, 1)
    cv_ref[...] = jnp.where(lane == 3, 1.0, cv)


def _edge_mlp(g, d, ea, wea, wrad, be1, we2, be2, wa, ba, wc1, bc1, wc2):
    eb = 2000
    grid = E // eb
    full = lambda shp: pl.BlockSpec(shp, lambda i: tuple(0 for _ in shp))
    return pl.pallas_call(
        _edge_body,
        grid=(grid,),
        in_specs=[
            pl.BlockSpec((eb, D), lambda i: (i, 0)),
            pl.BlockSpec((eb, XP), lambda i: (i, 0)),
            pl.BlockSpec((eb, 4), lambda i: (i, 0)),
            full((4, D)), full((1, D)), full((1, D)), full((D, D)),
            full((1, D)), full((D, 1)), full((1, 1)), full((D, D)),
            full((1, D)), full((D, 1)),
        ],
        out_specs=[
            pl.BlockSpec((eb, D), lambda i: (i, 0)),
            pl.BlockSpec((eb, XP), lambda i: (i, 0)),
        ],
        out_shape=[
            jax.ShapeDtypeStruct((E, D), _f32),
            jax.ShapeDtypeStruct((E, XP), _f32),
        ],
    )(g, d, ea, wea, wrad, be1, we2, be2, wa, ba, wc1, bc1, wc2)


@functools.partial(
    pl.kernel,
    out_type=(jax.ShapeDtypeStruct((NC, N, D), _f32),
              jax.ShapeDtypeStruct((NC, N, XP), _f32)),
    mesh=_mesh,
    scratch_types=[
        pltpu.VMEM((CH, D), _f32),
        pltpu.VMEM((CH, XP), _f32),
        pltpu.VMEM((CH,), jnp.int32),
        pltpu.VMEM_SHARED((N, D), _f32),
        pltpu.VMEM_SHARED((N, XP), _f32),
    ],
    compiler_params=_sc_params,
)
def _sc_scatter(m_hbm, cv_hbm, row_hbm, zh_hbm, zc_hbm, aggh_hbm, aggc_hbm,
                mbuf, cvbuf, idx, acch, accc):
    cid = lax.axis_index("c")
    sid = lax.axis_index("s")
    wid = sid * NC + cid
    rows = pl.ds(sid * RPS, RPS)
    pltpu.sync_copy(zh_hbm.at[rows], acch.at[rows])
    pltpu.sync_copy(zc_hbm.at[rows], accc.at[rows])
    plsc.subcore_barrier()

    @pl.loop(0, NCHUNK)
    def _chunk(ci):
        base = wid * EPW + ci * CH
        pltpu.sync_copy(row_hbm.at[pl.ds(base, CH)], idx)
        pltpu.sync_copy(m_hbm.at[pl.ds(base, CH)], mbuf)
        pltpu.sync_copy(cv_hbm.at[pl.ds(base, CH)], cvbuf)
        pltpu.sync_copy(mbuf, acch.at[idx], add=True)
        pltpu.sync_copy(cvbuf, accc.at[idx], add=True)

    plsc.subcore_barrier()
    pltpu.sync_copy(acch.at[rows], aggh_hbm.at[cid, rows])
    pltpu.sync_copy(accc.at[rows], aggc_hbm.at[cid, rows])


def _node_body(h_ref, aggp_ref, cp_ref, xp_ref, wn1h_ref, wn1a_ref, bn1_ref,
               wn2_ref, bn2_ref, ho_ref, xo_ref):
    h = h_ref[...]
    agg = aggp_ref[0] + aggp_ref[1]
    u = jax.nn.silu(jnp.dot(h, wn1h_ref[...], preferred_element_type=_f32)
                    + jnp.dot(agg, wn1a_ref[...], preferred_element_type=_f32)
                    + bn1_ref[...])
    ho_ref[...] = h + jnp.dot(u, wn2_ref[...], preferred_element_type=_f32) \
        + bn2_ref[...]
    s = cp_ref[0] + cp_ref[1]
    cnt = jnp.maximum(s[:, 3:4], 1.0)
    lane = lax.broadcasted_iota(jnp.int32, s.shape, 1)
    xo_ref[...] = xp_ref[...] + jnp.where(lane < 3, s, 0.0) / cnt


def _node_update(h, aggp, cp, xp, wn1h, wn1a, bn1, wn2, bn2):
    nb = 1000
    grid = N // nb
    full = lambda shp: pl.BlockSpec(shp, lambda i: tuple(0 for _ in shp))
    return pl.pallas_call(
        _node_body,
        grid=(grid,),
        in_specs=[
            pl.BlockSpec((nb, D), lambda i: (i, 0)),
            pl.BlockSpec((NC, nb, D), lambda i: (0, i, 0)),
            pl.BlockSpec((NC, nb, XP), lambda i: (0, i, 0)),
            pl.BlockSpec((nb, XP), lambda i: (i, 0)),
            full((D, D)), full((D, D)), full((1, D)), full((D, D)),
            full((1, D)),
        ],
        out_specs=[
            pl.BlockSpec((nb, D), lambda i: (i, 0)),
            pl.BlockSpec((nb, XP), lambda i: (i, 0)),
        ],
        out_shape=[
            jax.ShapeDtypeStruct((N, D), _f32),
            jax.ShapeDtypeStruct((N, XP), _f32),
        ],
    )(h, aggp, cp, xp, wn1h, wn1a, bn1, wn2, bn2)


def kernel(h, x, edge_index, edge_attr, W_e1, b_e1, W_e2, b_e2, W_n1, b_n1,
           W_n2, b_n2, W_c1, b_c1, W_c2, W_a, b_a):
    row = edge_index[0]
    col = edge_index[1]
    xp = jnp.pad(x, ((0, 0), (0, XP - 3)))

    whr = W_e1[:D]
    whc = W_e1[D:2 * D]
    wrad = W_e1[2 * D:2 * D + 1]
    wea = W_e1[2 * D + 1:]

    tr, tc = _make_tables(h, xp, whr, whc)
    g, d = _sc_gather(tr, tc, row, col)
    m, cv = _edge_mlp(g, d, edge_attr, wea, wrad, b_e1.reshape(1, D),
                      W_e2, b_e2.reshape(1, D), W_a, b_a.reshape(1, 1),
                      W_c1, b_c1.reshape(1, D), W_c2)
    zh = jnp.zeros((N, D), _f32)
    zc = jnp.zeros((N, XP), _f32)
    aggp, cp = _sc_scatter(m, cv, row, zh, zc)
    ho, xo = _node_update(h, aggp, cp, xp, W_n1[:D], W_n1[D:],
                          b_n1.reshape(1, D), W_n2, b_n2.reshape(1, D))
    return ho, xo[:, :3]


# speedup vs baseline: 4.0941x; 1.3408x over previous
import functools

import jax
import jax.numpy as jnp
from jax import lax
from jax.experimental import pallas as pl
from jax.experimental.pallas import tpu as pltpu
from jax.experimental.pallas import tpu_sc as plsc

N = 10000
E = 320000
D = 128
XP = 16
TW = D + XP

NC, NS, L = 2, 16, 16
NW = NC * NS
EPW = E // NW
CH = 80
NCHUNK = EPW // CH
RPS = N // NS

_f32 = jnp.float32
_mesh = plsc.VectorSubcoreMesh(core_axis_name="c", subcore_axis_name="s")
_sc_params = pltpu.CompilerParams(use_tc_tiling_on_sc=False)


def _tables_body(h_ref, xp_ref, whr_ref, whc_ref, tr_ref, tc_ref):
    h = h_ref[...]
    xp = xp_ref[...]
    tr_ref[...] = jnp.concatenate(
        [jnp.dot(h, whr_ref[...], preferred_element_type=_f32), xp], axis=1)
    tc_ref[...] = jnp.concatenate(
        [jnp.dot(h, whc_ref[...], preferred_element_type=_f32), xp], axis=1)


def _make_tables(h, xp, whr, whc):
    nb = 1000
    grid = N // nb
    return pl.pallas_call(
        _tables_body,
        grid=(grid,),
        in_specs=[
            pl.BlockSpec((nb, D), lambda i: (i, 0)),
            pl.BlockSpec((nb, XP), lambda i: (i, 0)),
            pl.BlockSpec((D, D), lambda i: (0, 0)),
            pl.BlockSpec((D, D), lambda i: (0, 0)),
        ],
        out_specs=[
            pl.BlockSpec((nb, TW), lambda i: (i, 0)),
            pl.BlockSpec((nb, TW), lambda i: (i, 0)),
        ],
        out_shape=[
            jax.ShapeDtypeStruct((N, TW), _f32),
            jax.ShapeDtypeStruct((N, TW), _f32),
        ],
    )(h, xp, whr, whc)


@functools.partial(
    pl.kernel,
    out_type=(jax.ShapeDtypeStruct((E, D), _f32),
              jax.ShapeDtypeStruct((E, XP), _f32)),
    mesh=_mesh,
    scratch_types=[
        pltpu.VMEM((2, CH), jnp.int32),
        pltpu.VMEM((2, CH), jnp.int32),
        pltpu.VMEM((2, CH, TW), _f32),
        pltpu.VMEM((2, CH, TW), _f32),
        pltpu.VMEM((2, CH, D), _f32),
        pltpu.VMEM((2, CH, XP), _f32),
        pltpu.SemaphoreType.DMA,
        pltpu.SemaphoreType.DMA,
        pltpu.SemaphoreType.DMA,
        pltpu.SemaphoreType.DMA,
        pltpu.SemaphoreType.DMA,
        pltpu.SemaphoreType.DMA,
    ],
    compiler_params=_sc_params,
)
def _sc_gather(tr_hbm, tc_hbm, row_hbm, col_hbm, g_hbm, d_hbm,
               idxr, idxc, abuf, bbuf, gbuf, dbuf,
               sa0, sa1, sb0, sb1, w0, w1):
    wid = lax.axis_index("s") * NC + lax.axis_index("c")
    sa = (sa0, sa1)
    sb = (sb0, sb1)
    ws = (w0, w1)

    def ebase(ci):
        return wid * EPW + ci * CH

    def issue(ci, b):
        base = ebase(ci)
        pltpu.sync_copy(row_hbm.at[pl.ds(base, CH)], idxr.at[b])
        pltpu.sync_copy(col_hbm.at[pl.ds(base, CH)], idxc.at[b])
        pltpu.async_copy(tr_hbm.at[idxr.at[b]], abuf.at[b], sa[b])
        pltpu.async_copy(tc_hbm.at[idxc.at[b]], bbuf.at[b], sb[b])

    def wait_gather(b):
        pltpu.make_async_copy(tr_hbm.at[idxr.at[b]], abuf.at[b], sa[b]).wait()
        pltpu.make_async_copy(tc_hbm.at[idxc.at[b]], bbuf.at[b], sb[b]).wait()

    def wait_write(ci, b):
        base = ebase(ci)
        pltpu.make_async_copy(gbuf.at[b], g_hbm.at[pl.ds(base, CH)],
                              ws[b]).wait()
        pltpu.make_async_copy(dbuf.at[b], d_hbm.at[pl.ds(base, CH)],
                              ws[b]).wait()

    def compute(b):
        @pl.loop(0, CH)
        def _row(i):
            for j in range(D // L):
                sl = pl.ds(j * L, L)
                gbuf[b, i, sl] = abuf[b, i, sl] + bbuf[b, i, sl]
            xs = pl.ds(D, L)
            dbuf[b, i, pl.ds(0, L)] = abuf[b, i, xs] - bbuf[b, i, xs]

    issue(0, 0)
    issue(1, 1)

    @pl.loop(0, NCHUNK - 1, step=2)
    def _chunk(ci):
        for b in (0, 1):
            cur = ci + b
            wait_gather(b)

            @pl.when(cur >= 2)
            def _():
                wait_write(cur - 2, b)

            compute(b)

            @pl.when(cur + 2 < NCHUNK)
            def _():
                issue(cur + 2, b)

            base = ebase(cur)
            pltpu.async_copy(gbuf.at[b], g_hbm.at[pl.ds(base, CH)], ws[b])
            pltpu.async_copy(dbuf.at[b], d_hbm.at[pl.ds(base, CH)], ws[b])

    last = NCHUNK - 1
    wait_gather(0)
    wait_write(last - 2, 0)
    compute(0)
    base = ebase(last)
    pltpu.sync_copy(gbuf.at[0], g_hbm.at[pl.ds(base, CH)])
    pltpu.sync_copy(dbuf.at[0], d_hbm.at[pl.ds(base, CH)])
    wait_write(last - 1, 1)


def _edge_body(g_ref, d_ref, ea_ref, wea_ref, wrad_ref, be1_ref, we2_ref,
               be2_ref, wa_ref, ba_ref, wc1_ref, bc1_ref, wc2_ref,
               m_ref, cv_ref):
    g = g_ref[...]
    d = d_ref[...]
    ea = ea_ref[...]
    radial = jnp.sum(d * d, axis=1, keepdims=True)
    pre = (g + jnp.dot(ea, wea_ref[...], preferred_element_type=_f32)
           + radial * wrad_ref[...] + be1_ref[...])
    m1 = jax.nn.silu(pre)
    m2 = jax.nn.silu(jnp.dot(m1, we2_ref[...], preferred_element_type=_f32)
                     + be2_ref[...])
    att = jax.nn.sigmoid(jnp.dot(m2, wa_ref[...], preferred_element_type=_f32)
                         + ba_ref[...])
    m = m2 * att
    m_ref[...] = m
    cu = jnp.dot(jax.nn.silu(jnp.dot(m, wc1_ref[...],
                                     preferred_element_type=_f32)
                             + bc1_ref[...]),
                 wc2_ref[...], preferred_element_type=_f32)
    cv = d * cu
    lane = lax.broadcasted_iota(jnp.int32, cv.shape, 1)
    cv_ref[...] = jnp.where(lane == 3, 1.0, cv)


def _edge_mlp(g, d, ea, wea, wrad, be1, we2, be2, wa, ba, wc1, bc1, wc2):
    eb = 2000
    grid = E // eb
    full = lambda shp: pl.BlockSpec(shp, lambda i: tuple(0 for _ in shp))
    return pl.pallas_call(
        _edge_body,
        grid=(grid,),
        in_specs=[
            pl.BlockSpec((eb, D), lambda i: (i, 0)),
            pl.BlockSpec((eb, XP), lambda i: (i, 0)),
            pl.BlockSpec((eb, 4), lambda i: (i, 0)),
            full((4, D)), full((1, D)), full((1, D)), full((D, D)),
            full((1, D)), full((D, 1)), full((1, 1)), full((D, D)),
            full((1, D)), full((D, 1)),
        ],
        out_specs=[
            pl.BlockSpec((eb, D), lambda i: (i, 0)),
            pl.BlockSpec((eb, XP), lambda i: (i, 0)),
        ],
        out_shape=[
            jax.ShapeDtypeStruct((E, D), _f32),
            jax.ShapeDtypeStruct((E, XP), _f32),
        ],
    )(g, d, ea, wea, wrad, be1, we2, be2, wa, ba, wc1, bc1, wc2)


@functools.partial(
    pl.kernel,
    out_type=(jax.ShapeDtypeStruct((NC, N, D), _f32),
              jax.ShapeDtypeStruct((NC, N, XP), _f32)),
    mesh=_mesh,
    scratch_types=[
        pltpu.VMEM((2, CH, D), _f32),
        pltpu.VMEM((2, CH, XP), _f32),
        pltpu.VMEM((2, CH), jnp.int32),
        pltpu.VMEM_SHARED((N, D), _f32),
        pltpu.VMEM_SHARED((N, XP), _f32),
        pltpu.SemaphoreType.DMA,
        pltpu.SemaphoreType.DMA,
    ],
    compiler_params=_sc_params,
)
def _sc_scatter(m_hbm, cv_hbm, row_hbm, zh_hbm, zc_hbm, aggh_hbm, aggc_hbm,
                mbuf, cvbuf, idx, acch, accc, l0, l1):
    cid = lax.axis_index("c")
    sid = lax.axis_index("s")
    wid = sid * NC + cid
    rows = pl.ds(sid * RPS, RPS)
    ls = (l0, l1)

    def ebase(ci):
        return wid * EPW + ci * CH

    def issue(ci, b):
        base = ebase(ci)
        pltpu.async_copy(row_hbm.at[pl.ds(base, CH)], idx.at[b], ls[b])
        pltpu.async_copy(m_hbm.at[pl.ds(base, CH)], mbuf.at[b], ls[b])
        pltpu.async_copy(cv_hbm.at[pl.ds(base, CH)], cvbuf.at[b], ls[b])

    def wait_loads(ci, b):
        base = ebase(ci)
        pltpu.make_async_copy(row_hbm.at[pl.ds(base, CH)], idx.at[b],
                              ls[b]).wait()
        pltpu.make_async_copy(m_hbm.at[pl.ds(base, CH)], mbuf.at[b],
                              ls[b]).wait()
        pltpu.make_async_copy(cv_hbm.at[pl.ds(base, CH)], cvbuf.at[b],
                              ls[b]).wait()

    issue(0, 0)
    issue(1, 1)
    pltpu.sync_copy(zh_hbm.at[rows], acch.at[rows])
    pltpu.sync_copy(zc_hbm.at[rows], accc.at[rows])
    plsc.subcore_barrier()

    @pl.loop(0, NCHUNK - 1, step=2)
    def _chunk(ci):
        for b in (0, 1):
            cur = ci + b
            wait_loads(cur, b)
            pltpu.sync_copy(mbuf.at[b], acch.at[idx.at[b]], add=True)
            pltpu.sync_copy(cvbuf.at[b], accc.at[idx.at[b]], add=True)

            @pl.when(cur + 2 < NCHUNK)
            def _():
                issue(cur + 2, b)

    last = NCHUNK - 1
    wait_loads(last, 0)
    pltpu.sync_copy(mbuf.at[0], acch.at[idx.at[0]], add=True)
    pltpu.sync_copy(cvbuf.at[0], accc.at[idx.at[0]], add=True)

    plsc.subcore_barrier()
    pltpu.sync_copy(acch.at[rows], aggh_hbm.at[cid, rows])
    pltpu.sync_copy(accc.at[rows], aggc_hbm.at[cid, rows])


def _node_body(h_ref, aggp_ref, cp_ref, xp_ref, wn1h_ref, wn1a_ref, bn1_ref,
               wn2_ref, bn2_ref, ho_ref, xo_ref):
    h = h_ref[...]
    agg = aggp_ref[0] + aggp_ref[1]
    u = jax.nn.silu(jnp.dot(h, wn1h_ref[...], preferred_element_type=_f32)
                    + jnp.dot(agg, wn1a_ref[...], preferred_element_type=_f32)
                    + bn1_ref[...])
    ho_ref[...] = h + jnp.dot(u, wn2_ref[...], preferred_element_type=_f32) \
        + bn2_ref[...]
    s = cp_ref[0] + cp_ref[1]
    cnt = jnp.maximum(s[:, 3:4], 1.0)
    lane = lax.broadcasted_iota(jnp.int32, s.shape, 1)
    xo_ref[...] = xp_ref[...] + jnp.where(lane < 3, s, 0.0) / cnt


def _node_update(h, aggp, cp, xp, wn1h, wn1a, bn1, wn2, bn2):
    nb = 1000
    grid = N // nb
    full = lambda shp: pl.BlockSpec(shp, lambda i: tuple(0 for _ in shp))
    return pl.pallas_call(
        _node_body,
        grid=(grid,),
        in_specs=[
            pl.BlockSpec((nb, D), lambda i: (i, 0)),
            pl.BlockSpec((NC, nb, D), lambda i: (0, i, 0)),
            pl.BlockSpec((NC, nb, XP), lambda i: (0, i, 0)),
            pl.BlockSpec((nb, XP), lambda i: (i, 0)),
            full((D, D)), full((D, D)), full((1, D)), full((D, D)),
            full((1, D)),
        ],
        out_specs=[
            pl.BlockSpec((nb, D), lambda i: (i, 0)),
            pl.BlockSpec((nb, XP), lambda i: (i, 0)),
        ],
        out_shape=[
            jax.ShapeDtypeStruct((N, D), _f32),
            jax.ShapeDtypeStruct((N, XP), _f32),
        ],
    )(h, aggp, cp, xp, wn1h, wn1a, bn1, wn2, bn2)


def kernel(h, x, edge_index, edge_attr, W_e1, b_e1, W_e2, b_e2, W_n1, b_n1,
           W_n2, b_n2, W_c1, b_c1, W_c2, W_a, b_a):
    row = edge_index[0]
    col = edge_index[1]
    xp = jnp.pad(x, ((0, 0), (0, XP - 3)))

    whr = W_e1[:D]
    whc = W_e1[D:2 * D]
    wrad = W_e1[2 * D:2 * D + 1]
    wea = W_e1[2 * D + 1:]

    tr, tc = _make_tables(h, xp, whr, whc)
    g, d = _sc_gather(tr, tc, row, col)
    m, cv = _edge_mlp(g, d, edge_attr, wea, wrad, b_e1.reshape(1, D),
                      W_e2, b_e2.reshape(1, D), W_a, b_a.reshape(1, 1),
                      W_c1, b_c1.reshape(1, D), W_c2)
    zh = jnp.zeros((N, D), _f32)
    zc = jnp.zeros((N, XP), _f32)
    aggp, cp = _sc_scatter(m, cv, row, zh, zc)
    ho, xo = _node_update(h, aggp, cp, xp, W_n1[:D], W_n1[D:],
                          b_n1.reshape(1, D), W_n2, b_n2.reshape(1, D))
    return ho, xo[:, :3]


# speedup vs baseline: 5.4683x; 1.3357x over previous
import functools

import jax
import jax.numpy as jnp
from jax import lax
from jax.experimental import pallas as pl
from jax.experimental.pallas import tpu as pltpu
from jax.experimental.pallas import tpu_sc as plsc

N = 10000
E = 320000
D = 128
XP = 16
TW = D + XP

NC, NS, L = 2, 16, 16
NW = NC * NS
NSLICE = 5
ES = E // NSLICE
EPW = ES // NW
CH = 80
NCHUNK = EPW // CH
RPS = N // NS

_f32 = jnp.float32
_mesh = plsc.VectorSubcoreMesh(core_axis_name="c", subcore_axis_name="s")
_sc_params = pltpu.CompilerParams(use_tc_tiling_on_sc=False)


def _tables_body(h_ref, xp_ref, whr_ref, whc_ref, tr_ref, tc_ref):
    h = h_ref[...]
    xp = xp_ref[...]
    tr_ref[...] = jnp.concatenate(
        [jnp.dot(h, whr_ref[...], preferred_element_type=_f32), xp], axis=1)
    tc_ref[...] = jnp.concatenate(
        [jnp.dot(h, whc_ref[...], preferred_element_type=_f32), xp], axis=1)


def _make_tables(h, xp, whr, whc):
    nb = 1000
    grid = N // nb
    return pl.pallas_call(
        _tables_body,
        grid=(grid,),
        in_specs=[
            pl.BlockSpec((nb, D), lambda i: (i, 0)),
            pl.BlockSpec((nb, XP), lambda i: (i, 0)),
            pl.BlockSpec((D, D), lambda i: (0, 0)),
            pl.BlockSpec((D, D), lambda i: (0, 0)),
        ],
        out_specs=[
            pl.BlockSpec((nb, TW), lambda i: (i, 0)),
            pl.BlockSpec((nb, TW), lambda i: (i, 0)),
        ],
        out_shape=[
            jax.ShapeDtypeStruct((N, TW), _f32),
            jax.ShapeDtypeStruct((N, TW), _f32),
        ],
    )(h, xp, whr, whc)


@functools.partial(
    pl.kernel,
    out_type=(jax.ShapeDtypeStruct((ES, D), _f32),
              jax.ShapeDtypeStruct((ES, XP), _f32)),
    mesh=_mesh,
    scratch_types=[
        pltpu.VMEM((2, CH), jnp.int32),
        pltpu.VMEM((2, CH), jnp.int32),
        pltpu.VMEM((2, CH, TW), _f32),
        pltpu.VMEM((2, CH, TW), _f32),
        pltpu.VMEM((2, CH, D), _f32),
        pltpu.VMEM((2, CH, XP), _f32),
        pltpu.SemaphoreType.DMA,
        pltpu.SemaphoreType.DMA,
        pltpu.SemaphoreType.DMA,
        pltpu.SemaphoreType.DMA,
        pltpu.SemaphoreType.DMA,
        pltpu.SemaphoreType.DMA,
    ],
    compiler_params=_sc_params,
)
def _sc_gather(tr_hbm, tc_hbm, row_hbm, col_hbm, g_hbm, d_hbm,
               idxr, idxc, abuf, bbuf, gbuf, dbuf,
               sa0, sa1, sb0, sb1, w0, w1):
    wid = lax.axis_index("s") * NC + lax.axis_index("c")
    sa = (sa0, sa1)
    sb = (sb0, sb1)
    ws = (w0, w1)

    def ebase(ci):
        return wid * EPW + ci * CH

    def issue(ci, b):
        base = ebase(ci)
        pltpu.sync_copy(row_hbm.at[pl.ds(base, CH)], idxr.at[b])
        pltpu.sync_copy(col_hbm.at[pl.ds(base, CH)], idxc.at[b])
        pltpu.async_copy(tr_hbm.at[idxr.at[b]], abuf.at[b], sa[b])
        pltpu.async_copy(tc_hbm.at[idxc.at[b]], bbuf.at[b], sb[b])

    def wait_gather(b):
        pltpu.make_async_copy(tr_hbm.at[idxr.at[b]], abuf.at[b], sa[b]).wait()
        pltpu.make_async_copy(tc_hbm.at[idxc.at[b]], bbuf.at[b], sb[b]).wait()

    def wait_write(ci, b):
        base = ebase(ci)
        pltpu.make_async_copy(gbuf.at[b], g_hbm.at[pl.ds(base, CH)],
                              ws[b]).wait()
        pltpu.make_async_copy(dbuf.at[b], d_hbm.at[pl.ds(base, CH)],
                              ws[b]).wait()

    def compute(b):
        @pl.loop(0, CH)
        def _row(i):
            for j in range(D // L):
                sl = pl.ds(j * L, L)
                gbuf[b, i, sl] = abuf[b, i, sl] + bbuf[b, i, sl]
            xs = pl.ds(D, L)
            dbuf[b, i, pl.ds(0, L)] = abuf[b, i, xs] - bbuf[b, i, xs]

    issue(0, 0)
    issue(1, 1)

    @pl.loop(0, NCHUNK - 1, step=2)
    def _chunk(ci):
        for b in (0, 1):
            cur = ci + b
            wait_gather(b)

            @pl.when(cur >= 2)
            def _():
                wait_write(cur - 2, b)

            compute(b)

            @pl.when(cur + 2 < NCHUNK)
            def _():
                issue(cur + 2, b)

            base = ebase(cur)
            pltpu.async_copy(gbuf.at[b], g_hbm.at[pl.ds(base, CH)], ws[b])
            pltpu.async_copy(dbuf.at[b], d_hbm.at[pl.ds(base, CH)], ws[b])

    last = NCHUNK - 1
    wait_gather(0)
    wait_write(last - 2, 0)
    compute(0)
    base = ebase(last)
    pltpu.sync_copy(gbuf.at[0], g_hbm.at[pl.ds(base, CH)])
    pltpu.sync_copy(dbuf.at[0], d_hbm.at[pl.ds(base, CH)])
    wait_write(last - 1, 1)


def _edge_body(g_ref, d_ref, ea_ref, wea_ref, wrad_ref, be1_ref, we2_ref,
               be2_ref, wa_ref, ba_ref, wc1_ref, bc1_ref, wc2_ref,
               m_ref, cv_ref):
    g = g_ref[...]
    d = d_ref[...]
    ea = ea_ref[...]
    radial = jnp.sum(d * d, axis=1, keepdims=True)
    pre = (g + jnp.dot(ea, wea_ref[...], preferred_element_type=_f32)
           + radial * wrad_ref[...] + be1_ref[...])
    m1 = jax.nn.silu(pre)
    m2 = jax.nn.silu(jnp.dot(m1, we2_ref[...], preferred_element_type=_f32)
                     + be2_ref[...])
    att = jax.nn.sigmoid(jnp.dot(m2, wa_ref[...], preferred_element_type=_f32)
                         + ba_ref[...])
    m = m2 * att
    m_ref[...] = m
    cu = jnp.dot(jax.nn.silu(jnp.dot(m, wc1_ref[...],
                                     preferred_element_type=_f32)
                             + bc1_ref[...]),
                 wc2_ref[...], preferred_element_type=_f32)
    cv = d * cu
    lane = lax.broadcasted_iota(jnp.int32, cv.shape, 1)
    cv_ref[...] = jnp.where(lane == 3, 1.0, cv)


def _edge_mlp(g, d, ea, wea, wrad, be1, we2, be2, wa, ba, wc1, bc1, wc2):
    eb = 2000
    grid = ES // eb
    full = lambda shp: pl.BlockSpec(shp, lambda i: tuple(0 for _ in shp))
    return pl.pallas_call(
        _edge_body,
        grid=(grid,),
        in_specs=[
            pl.BlockSpec((eb, D), lambda i: (i, 0)),
            pl.BlockSpec((eb, XP), lambda i: (i, 0)),
            pl.BlockSpec((eb, 4), lambda i: (i, 0)),
            full((4, D)), full((1, D)), full((1, D)), full((D, D)),
            full((1, D)), full((D, 1)), full((1, 1)), full((D, D)),
            full((1, D)), full((D, 1)),
        ],
        out_specs=[
            pl.BlockSpec((eb, D), lambda i: (i, 0)),
            pl.BlockSpec((eb, XP), lambda i: (i, 0)),
        ],
        out_shape=[
            jax.ShapeDtypeStruct((ES, D), _f32),
            jax.ShapeDtypeStruct((ES, XP), _f32),
        ],
    )(g, d, ea, wea, wrad, be1, we2, be2, wa, ba, wc1, bc1, wc2)


@functools.partial(
    pl.kernel,
    out_type=(jax.ShapeDtypeStruct((NC, N, D), _f32),
              jax.ShapeDtypeStruct((NC, N, XP), _f32)),
    mesh=_mesh,
    scratch_types=[
        pltpu.VMEM((2, CH, D), _f32),
        pltpu.VMEM((2, CH, XP), _f32),
        pltpu.VMEM((2, CH), jnp.int32),
        pltpu.VMEM_SHARED((N, D), _f32),
        pltpu.VMEM_SHARED((N, XP), _f32),
        pltpu.SemaphoreType.DMA,
        pltpu.SemaphoreType.DMA,
    ],
    compiler_params=_sc_params,
)
def _sc_scatter(m0, m1, m2, m3, m4, cv0, cv1, cv2, cv3, cv4, row_hbm,
                zh_hbm, zc_hbm, aggh_hbm, aggc_hbm,
                mbuf, cvbuf, idx, acch, accc, l0, l1):
    cid = lax.axis_index("c")
    sid = lax.axis_index("s")
    wid = sid * NC + cid
    rows = pl.ds(sid * RPS, RPS)
    ls = (l0, l1)
    m_s = (m0, m1, m2, m3, m4)
    cv_s = (cv0, cv1, cv2, cv3, cv4)

    def issue(s, ci, b):
        base = wid * EPW + ci * CH
        pltpu.async_copy(row_hbm.at[pl.ds(s * ES + base, CH)], idx.at[b],
                         ls[b])
        pltpu.async_copy(m_s[s].at[pl.ds(base, CH)], mbuf.at[b], ls[b])
        pltpu.async_copy(cv_s[s].at[pl.ds(base, CH)], cvbuf.at[b], ls[b])

    def wait_loads(s, ci, b):
        base = wid * EPW + ci * CH
        pltpu.make_async_copy(row_hbm.at[pl.ds(s * ES + base, CH)],
                              idx.at[b], ls[b]).wait()
        pltpu.make_async_copy(m_s[s].at[pl.ds(base, CH)], mbuf.at[b],
                              ls[b]).wait()
        pltpu.make_async_copy(cv_s[s].at[pl.ds(base, CH)], cvbuf.at[b],
                              ls[b]).wait()

    def scat(b):
        pltpu.sync_copy(mbuf.at[b], acch.at[idx.at[b]], add=True)
        pltpu.sync_copy(cvbuf.at[b], accc.at[idx.at[b]], add=True)

    issue(0, 0, 0)
    issue(0, 1, 1)
    pltpu.sync_copy(zh_hbm.at[rows], acch.at[rows])
    pltpu.sync_copy(zc_hbm.at[rows], accc.at[rows])
    plsc.subcore_barrier()

    for s in range(NSLICE):
        @pl.loop(0, NCHUNK - 1, step=2)
        def _chunk(ci, s=s):
            for b in (0, 1):
                bb = (b + s) % 2
                cur = ci + b
                wait_loads(s, cur, bb)
                scat(bb)
                nxt = cur + 2
                if s + 1 < NSLICE:
                    @pl.when(nxt < NCHUNK)
                    def _():
                        issue(s, nxt, bb)

                    @pl.when(nxt >= NCHUNK)
                    def _():
                        issue(s + 1, nxt - NCHUNK, bb)
                else:
                    @pl.when(nxt < NCHUNK)
                    def _():
                        issue(s, nxt, bb)

        last = NCHUNK - 1
        bb = s % 2
        wait_loads(s, last, bb)
        scat(bb)
        if s + 1 < NSLICE:
            issue(s + 1, 1, bb)

    plsc.subcore_barrier()
    pltpu.sync_copy(acch.at[rows], aggh_hbm.at[cid, rows])
    pltpu.sync_copy(accc.at[rows], aggc_hbm.at[cid, rows])


def _node_body(h_ref, aggp_ref, cp_ref, xp_ref, wn1h_ref, wn1a_ref, bn1_ref,
               wn2_ref, bn2_ref, ho_ref, xo_ref):
    h = h_ref[...]
    agg = aggp_ref[0] + aggp_ref[1]
    u = jax.nn.silu(jnp.dot(h, wn1h_ref[...], preferred_element_type=_f32)
                    + jnp.dot(agg, wn1a_ref[...], preferred_element_type=_f32)
                    + bn1_ref[...])
    ho_ref[...] = h + jnp.dot(u, wn2_ref[...], preferred_element_type=_f32) \
        + bn2_ref[...]
    s = cp_ref[0] + cp_ref[1]
    cnt = jnp.maximum(s[:, 3:4], 1.0)
    lane = lax.broadcasted_iota(jnp.int32, s.shape, 1)
    xo_ref[...] = xp_ref[...] + jnp.where(lane < 3, s, 0.0) / cnt


def _node_update(h, aggp, cp, xp, wn1h, wn1a, bn1, wn2, bn2):
    nb = 1000
    grid = N // nb
    full = lambda shp: pl.BlockSpec(shp, lambda i: tuple(0 for _ in shp))
    return pl.pallas_call(
        _node_body,
        grid=(grid,),
        in_specs=[
            pl.BlockSpec((nb, D), lambda i: (i, 0)),
            pl.BlockSpec((NC, nb, D), lambda i: (0, i, 0)),
            pl.BlockSpec((NC, nb, XP), lambda i: (0, i, 0)),
            pl.BlockSpec((nb, XP), lambda i: (i, 0)),
            full((D, D)), full((D, D)), full((1, D)), full((D, D)),
            full((1, D)),
        ],
        out_specs=[
            pl.BlockSpec((nb, D), lambda i: (i, 0)),
            pl.BlockSpec((nb, XP), lambda i: (i, 0)),
        ],
        out_shape=[
            jax.ShapeDtypeStruct((N, D), _f32),
            jax.ShapeDtypeStruct((N, XP), _f32),
        ],
    )(h, aggp, cp, xp, wn1h, wn1a, bn1, wn2, bn2)


def kernel(h, x, edge_index, edge_attr, W_e1, b_e1, W_e2, b_e2, W_n1, b_n1,
           W_n2, b_n2, W_c1, b_c1, W_c2, W_a, b_a):
    row = edge_index[0]
    col = edge_index[1]
    xp = jnp.pad(x, ((0, 0), (0, XP - 3)))

    whr = W_e1[:D]
    whc = W_e1[D:2 * D]
    wrad = W_e1[2 * D:2 * D + 1]
    wea = W_e1[2 * D + 1:]

    tr, tc = _make_tables(h, xp, whr, whc)
    ms, cvs = [], []
    for s in range(NSLICE):
        sl = slice(s * ES, (s + 1) * ES)
        g, d = _sc_gather(tr, tc, row[sl], col[sl])
        m, cv = _edge_mlp(g, d, edge_attr[sl], wea, wrad, b_e1.reshape(1, D),
                          W_e2, b_e2.reshape(1, D), W_a, b_a.reshape(1, 1),
                          W_c1, b_c1.reshape(1, D), W_c2)
        ms.append(m)
        cvs.append(cv)
    zh = jnp.zeros((N, D), _f32)
    zc = jnp.zeros((N, XP), _f32)
    aggp, cp = _sc_scatter(*ms, *cvs, row, zh, zc)
    ho, xo = _node_update(h, aggp, cp, xp, W_n1[:D], W_n1[D:],
                          b_n1.reshape(1, D), W_n2, b_n2.reshape(1, D))
    return ho, xo[:, :3]
